# baseline (speedup 1.0000x reference)
import functools

import jax
import jax.numpy as jnp
from jax import lax
from jax.experimental import pallas as pl
from jax.experimental.pallas import tpu as pltpu
from jax.experimental.pallas import tpu_sc as plsc

_NC = 2
_NS = 16
_NW = _NC * _NS
_LANES = 16
_CHUNK = 400


@functools.partial(jax.jit, static_argnames=("n1", "n2", "d"))
def _sc_local_gather(a0, a1, a2, combo_flat, *, n1, n2, d):
    e = a0.shape[0]
    per_w = e // _NW
    chunk = _CHUNK
    assert per_w * _NW == e and per_w % chunk == 0 and chunk % _LANES == 0
    nchunk = per_w // chunk
    ngrp = chunk // _LANES
    m0 = n1 * n2 * d
    m1 = n2 * d
    ncw = combo_flat.shape[0]

    mesh = plsc.VectorSubcoreMesh(core_axis_name="c", subcore_axis_name="s")

    @functools.partial(
        pl.kernel,
        mesh=mesh,
        out_type=jax.ShapeDtypeStruct((e * d,), jnp.float32),
        compiler_params=pltpu.CompilerParams(
            use_tc_tiling_on_sc=False, needs_layout_passes=False),
        scratch_types=[
            pltpu.VMEM((ncw,), jnp.float32),
            pltpu.VMEM((chunk,), jnp.int32),
            pltpu.VMEM((chunk,), jnp.int32),
            pltpu.VMEM((chunk,), jnp.int32),
            pltpu.VMEM((chunk,), jnp.int32),
            pltpu.VMEM((chunk,), jnp.int32),
            pltpu.VMEM((chunk,), jnp.int32),
            pltpu.VMEM((chunk * d,), jnp.float32),
            pltpu.VMEM((chunk * d,), jnp.float32),
            pltpu.SemaphoreType.DMA,
            pltpu.SemaphoreType.DMA,
            pltpu.SemaphoreType.DMA,
            pltpu.SemaphoreType.DMA,
        ],
    )
    def k(a0_hbm, a1_hbm, a2_hbm, combo_hbm, out_hbm,
          combo_v, c0a, c1a, c2a, c0b, c1b, c2b, rows_a, rows_b,
          sca, scb, ssa, ssb):
        wid = lax.axis_index("s") * _NC + lax.axis_index("c")
        base = wid * per_w

        pltpu.sync_copy(combo_hbm, combo_v)

        cols = ((c0a, c1a, c2a), (c0b, c1b, c2b))
        rows = (rows_a, rows_b)
        sc = (sca, scb)
        ss = (ssa, ssb)
        st_iota = lax.iota(jnp.int32, _LANES) * d

        def col_descs(g, p):
            off = base + g * chunk
            cs = cols[p]
            return (
                pltpu.make_async_copy(a0_hbm.at[pl.ds(off, chunk)], cs[0], sc[p]),
                pltpu.make_async_copy(a1_hbm.at[pl.ds(off, chunk)], cs[1], sc[p]),
                pltpu.make_async_copy(a2_hbm.at[pl.ds(off, chunk)], cs[2], sc[p]),
            )

        def start_cols(g, p):
            for desc in col_descs(g, p):
                desc.start()

        def wait_cols(g, p):
            for desc in col_descs(g, p):
                desc.wait()

        def store_desc(g, p):
            return pltpu.make_async_copy(
                rows[p], out_hbm.at[pl.ds((base + g * chunk) * d, chunk * d)],
                ss[p])

        def compute(p):
            cs = cols[p]
            rv = rows[p]

            def grp_body(grp, carry):
                s = pl.ds(grp * _LANES, _LANES)
                ridx = cs[0][s] * m0 + cs[1][s] * m1 + cs[2][s] * d
                st0 = st_iota + grp * (_LANES * d)
                for j in range(d):
                    v = plsc.load_gather(combo_v, [ridx + j])
                    plsc.store_scatter(rv, [st0 + j], v)
                return carry

            lax.fori_loop(0, ngrp, grp_body, 0)

        def chunk_body(g, carry):
            wc = col_descs(g, 0)
            for desc in wc:
                desc.start()
            for desc in wc:
                desc.wait()
            compute(0)
            sd = store_desc(g, 0)
            sd.start()
            sd.wait()
            return carry

        lax.fori_loop(0, nchunk, chunk_body, 0)

    return k(a0, a1, a2, combo_flat)


def kernel(edge_attr, W0, W1, W2):
    ea = edge_attr.astype(jnp.int32)
    n1, n2 = W1.shape[0], W2.shape[0]
    d = W0.shape[1]
    combo = (W0[:, None, None, :] + W1[None, :, None, :]
             + W2[None, None, :, :]).reshape(-1)
    out = _sc_local_gather(ea[:, 0], ea[:, 1], ea[:, 2], combo,
                           n1=n1, n2=n2, d=d)
    return out.reshape(ea.shape[0], d)


# speedup vs baseline: 7.1190x; 7.1190x over previous
import functools

import jax
import jax.numpy as jnp
from jax import lax
from jax.experimental import pallas as pl
from jax.experimental.pallas import tpu as pltpu
from jax.experimental.pallas import tpu_sc as plsc

_NC = 2
_NS = 16
_NW = _NC * _NS
_LANES = 16
_KREP = 32


def _pick_chunk(per_w: int) -> int:
    for c in range(min(per_w, 400), 7, -1):
        if c % 8 == 0 and per_w % c == 0:
            return c
    return 0


@functools.partial(jax.jit, static_argnames=("n1", "n2"))
def _sc_combo_gather(a0, a1, a2, combo, *, n1, n2):
    e = a0.shape[0]
    d = combo.shape[1]
    per_w = e // _NW
    chunk = _pick_chunk(per_w)
    assert per_w * _NW == e and chunk, f"unsupported edge count {e}"
    nchunk = per_w // chunk
    m0 = n1 * n2
    n_combo = combo.shape[0] // (_NW * _KREP)

    mesh = plsc.VectorSubcoreMesh(core_axis_name="c", subcore_axis_name="s")

    @functools.partial(
        pl.kernel,
        mesh=mesh,
        out_type=jax.ShapeDtypeStruct((e, d), jnp.float32),
        scratch_types=[
            pltpu.VMEM((per_w,), jnp.int32),
            pltpu.VMEM((per_w,), jnp.int32),
            pltpu.VMEM((chunk, d), jnp.float32),
            pltpu.VMEM((chunk, d), jnp.float32),
            pltpu.SemaphoreType.DMA,
            pltpu.SemaphoreType.DMA,
            pltpu.SemaphoreType.DMA,
            pltpu.SemaphoreType.DMA,
        ],
    )
    def k(a0_hbm, a1_hbm, a2_hbm, combo_hbm, out_hbm,
          col_v, idx_v, rows_a, rows_b, sga, sgb, ssa, ssb):
        wid = lax.axis_index("s") * _NC + lax.axis_index("c")
        base = wid * per_w
        rep_base = wid * _KREP * n_combo

        ngrp = per_w // _LANES

        pltpu.sync_copy(a0_hbm.at[pl.ds(base, per_w)], col_v)

        lane_iota = lax.iota(jnp.int32, _LANES)

        def f0(i, c):
            s = pl.ds(i * _LANES, _LANES)
            rep = lax.rem(lane_iota + i, _KREP) * n_combo + rep_base
            idx_v[s] = col_v[s] * m0 + rep
            return c

        lax.fori_loop(0, ngrp, f0, 0)
        pltpu.sync_copy(a1_hbm.at[pl.ds(base, per_w)], col_v)

        def f1(i, c):
            s = pl.ds(i * _LANES, _LANES)
            idx_v[s] = idx_v[s] + col_v[s] * n2
            return c

        lax.fori_loop(0, ngrp, f1, 0)
        pltpu.sync_copy(a2_hbm.at[pl.ds(base, per_w)], col_v)

        def f2(i, c):
            s = pl.ds(i * _LANES, _LANES)
            idx_v[s] = idx_v[s] + col_v[s]
            return c

        lax.fori_loop(0, ngrp, f2, 0)

        rows = (rows_a, rows_b)
        sg = (sga, sgb)
        ss = (ssa, ssb)
        gath = {}
        stor = {}
        for g in range(nchunk):
            p = g % 2
            if g >= 2:
                stor[g - 2].wait()
            c = pltpu.make_async_copy(
                combo_hbm.at[idx_v.at[pl.ds(g * chunk, chunk)]],
                rows[p], sg[p])
            c.start()
            gath[g] = c
            if g >= 1:
                q = (g - 1) % 2
                gath[g - 1].wait()
                c = pltpu.make_async_copy(
                    rows[q], out_hbm.at[pl.ds(base + (g - 1) * chunk, chunk)],
                    ss[q])
                c.start()
                stor[g - 1] = c
        g = nchunk - 1
        gath[g].wait()
        c = pltpu.make_async_copy(
            rows[g % 2], out_hbm.at[pl.ds(base + g * chunk, chunk)],
            ss[g % 2])
        c.start()
        stor[g] = c
        stor[nchunk - 2].wait()
        stor[nchunk - 1].wait()

    return k(a0, a1, a2, combo)


def kernel(edge_attr, W0, W1, W2):
    ea = edge_attr.astype(jnp.int32)
    n1, n2 = W1.shape[0], W2.shape[0]
    combo = (W0[:, None, None, :] + W1[None, :, None, :]
             + W2[None, None, :, :]).reshape(-1, W0.shape[1])
    combo = jnp.tile(combo, (_NW * _KREP, 1))
    return _sc_combo_gather(ea[:, 0], ea[:, 1], ea[:, 2], combo,
                            n1=n1, n2=n2)


# speedup vs baseline: 7.4744x; 1.0499x over previous
import functools

import jax
import jax.numpy as jnp
from jax import lax
from jax.experimental import pallas as pl
from jax.experimental.pallas import tpu as pltpu
from jax.experimental.pallas import tpu_sc as plsc

_NC = 2
_NS = 16
_NW = _NC * _NS
_LANES = 16
_KREP = 16


def _pick_chunk(per_w: int) -> int:
    for c in range(min(per_w, 400), _LANES - 1, -1):
        if c % _LANES == 0 and per_w % c == 0:
            return c
    return 0


@functools.partial(jax.jit, static_argnames=("n1", "n2"))
def _sc_combo_gather(a0, a1, a2, combo, *, n1, n2):
    e = a0.shape[0]
    d = combo.shape[1]
    per_w = e // _NW
    chunk = _pick_chunk(per_w)
    assert per_w * _NW == e and chunk, f"unsupported edge count {e}"
    nchunk = per_w // chunk
    ngrp = chunk // _LANES
    m0 = n1 * n2
    n_combo = combo.shape[0] // (_NW * _KREP)

    mesh = plsc.VectorSubcoreMesh(core_axis_name="c", subcore_axis_name="s")

    @functools.partial(
        pl.kernel,
        mesh=mesh,
        out_type=jax.ShapeDtypeStruct((e, d), jnp.float32),
        scratch_types=[
            pltpu.VMEM((chunk,), jnp.int32),
            pltpu.VMEM((chunk,), jnp.int32),
            pltpu.VMEM((chunk,), jnp.int32),
            pltpu.VMEM((chunk,), jnp.int32),
            pltpu.VMEM((chunk,), jnp.int32),
            pltpu.VMEM((chunk,), jnp.int32),
            pltpu.VMEM((chunk,), jnp.int32),
            pltpu.VMEM((chunk,), jnp.int32),
            pltpu.VMEM((chunk, d), jnp.float32),
            pltpu.VMEM((chunk, d), jnp.float32),
            pltpu.SemaphoreType.DMA,
            pltpu.SemaphoreType.DMA,
            pltpu.SemaphoreType.DMA,
            pltpu.SemaphoreType.DMA,
            pltpu.SemaphoreType.DMA,
            pltpu.SemaphoreType.DMA,
        ],
    )
    def k(a0_hbm, a1_hbm, a2_hbm, combo_hbm, out_hbm,
          c0a, c0b, c1a, c1b, c2a, c2b, idx_a, idx_b, rows_a, rows_b,
          sca, scb, sga, sgb, ssa, ssb):
        wid = lax.axis_index("s") * _NC + lax.axis_index("c")
        base = wid * per_w
        rep_base = wid * _KREP * n_combo

        cols = ((c0a, c1a, c2a), (c0b, c1b, c2b))
        idx = (idx_a, idx_b)
        rows = (rows_a, rows_b)
        sc = (sca, scb)
        sg = (sga, sgb)
        ss = (ssa, ssb)
        lane_iota = lax.iota(jnp.int32, _LANES)

        def col_descs(g, p):
            off = base + g * chunk
            cs = cols[p]
            return (
                pltpu.make_async_copy(a0_hbm.at[pl.ds(off, chunk)], cs[0],
                                      sc[p]),
                pltpu.make_async_copy(a1_hbm.at[pl.ds(off, chunk)], cs[1],
                                      sc[p]),
                pltpu.make_async_copy(a2_hbm.at[pl.ds(off, chunk)], cs[2],
                                      sc[p]),
            )

        def fuse(g, p):
            c0, c1, c2 = cols[p]
            dst = idx[p]
            g0 = g * ngrp

            def body(i, c):
                s = pl.ds(i * _LANES, _LANES)
                rep = lax.rem(lane_iota + (g0 + i), _KREP) * n_combo
                dst[s] = c0[s] * m0 + c1[s] * n2 + c2[s] + (rep + rep_base)
                return c

            lax.fori_loop(0, ngrp, body, 0)

        gath = {}
        stor = {}
        for dd in col_descs(0, 0):
            dd.start()
        if nchunk > 1:
            for dd in col_descs(1, 1):
                dd.start()
        for g in range(nchunk):
            p = g % 2
            if g >= 2:
                stor[g - 2].wait()
            for dd in col_descs(g, p):
                dd.wait()
            fuse(g, p)
            if g + 2 < nchunk:
                for dd in col_descs(g + 2, p):
                    dd.start()
            c = pltpu.make_async_copy(
                combo_hbm.at[idx[p].at[:]], rows[p], sg[p])
            c.start()
            gath[g] = c
            if g >= 1:
                q = (g - 1) % 2
                gath[g - 1].wait()
                c = pltpu.make_async_copy(
                    rows[q], out_hbm.at[pl.ds(base + (g - 1) * chunk, chunk)],
                    ss[q])
                c.start()
                stor[g - 1] = c
        g = nchunk - 1
        gath[g].wait()
        c = pltpu.make_async_copy(
            rows[g % 2], out_hbm.at[pl.ds(base + g * chunk, chunk)],
            ss[g % 2])
        c.start()
        stor[g] = c
        if nchunk >= 2:
            stor[nchunk - 2].wait()
        stor[nchunk - 1].wait()

    return k(a0, a1, a2, combo)


def kernel(edge_attr, W0, W1, W2):
    ea = edge_attr.astype(jnp.int32)
    n1, n2 = W1.shape[0], W2.shape[0]
    combo = (W0[:, None, None, :] + W1[None, :, None, :]
             + W2[None, None, :, :]).reshape(-1, W0.shape[1])
    combo = jnp.tile(combo, (_NW * _KREP, 1))
    return _sc_combo_gather(ea[:, 0], ea[:, 1], ea[:, 2], combo,
                            n1=n1, n2=n2)
